# baseline (speedup 1.0000x reference)
import functools

import jax
import jax.numpy as jnp
from jax.experimental import pallas as pl
from jax.experimental.pallas import tpu as pltpu

_N = 20000
_TM = 2000


def _elu(v):
    return jnp.where(v > 0, v, jnp.exp(v) - 1.0)


def _accum(s_ref, cs):
    i = pl.program_id(0)

    @pl.when(i == 0)
    def _():
        s_ref[...] = cs

    @pl.when(i != 0)
    def _():
        s_ref[...] = s_ref[...] + cs


def _head_body(x_ref, wgt_ref, bg_ref, u_ref, s_ref):
    xb = x_ref[...]
    u = jnp.dot(xb.astype(jnp.bfloat16), wgt_ref[...],
                preferred_element_type=jnp.float32) + bg_ref[...]
    u_ref[...] = u.astype(u_ref.dtype)
    _accum(s_ref, jnp.sum(xb, axis=0, keepdims=True))


def _mid_body(sprev_ref, wlt_ref, u_ref, wgt_ref, bg_ref, uo_ref, s_ref,
              *, inv_n):
    m = (sprev_ref[...] * inv_n).astype(jnp.bfloat16)
    c = jnp.dot(m, wlt_ref[...], preferred_element_type=jnp.float32)
    h = _elu(u_ref[...].astype(jnp.float32) - c)
    u = jnp.dot(h.astype(jnp.bfloat16), wgt_ref[...],
                preferred_element_type=jnp.float32) + bg_ref[...]
    uo_ref[...] = u.astype(uo_ref.dtype)
    _accum(s_ref, jnp.sum(h, axis=0, keepdims=True))


def _tail_body(sprev_ref, wlt_ref, u_ref, o_ref, *, inv_n):
    m = (sprev_ref[...] * inv_n).astype(jnp.bfloat16)
    c = jnp.dot(m, wlt_ref[...], preferred_element_type=jnp.float32)
    o_ref[...] = _elu(u_ref[...] - c)


_PARAMS = pltpu.CompilerParams(dimension_semantics=("arbitrary",))


def _full(shape):
    return pl.BlockSpec(shape, lambda i: (0,) * len(shape))


def _head_call(x, wgt, bg):
    n, k = x.shape
    o = wgt.shape[1]
    return pl.pallas_call(
        _head_body,
        grid=(n // _TM,),
        in_specs=[pl.BlockSpec((_TM, k), lambda i: (i, 0)),
                  _full((k, o)), _full((1, o))],
        out_specs=[pl.BlockSpec((_TM, o), lambda i: (i, 0)),
                   _full((1, k))],
        out_shape=[jax.ShapeDtypeStruct((n, o), jnp.bfloat16),
                   jax.ShapeDtypeStruct((1, k), jnp.float32)],
        compiler_params=_PARAMS,
    )(x, wgt, bg)


def _mid_call(u, sprev, wlt, wgt, bg, out_dtype):
    n, k = u.shape
    pk = sprev.shape[1]
    o = wgt.shape[1]
    return pl.pallas_call(
        functools.partial(_mid_body, inv_n=1.0 / n),
        grid=(n // _TM,),
        in_specs=[_full((1, pk)), _full((pk, k)),
                  pl.BlockSpec((_TM, k), lambda i: (i, 0)),
                  _full((k, o)), _full((1, o))],
        out_specs=[pl.BlockSpec((_TM, o), lambda i: (i, 0)),
                   _full((1, k))],
        out_shape=[jax.ShapeDtypeStruct((n, o), out_dtype),
                   jax.ShapeDtypeStruct((1, k), jnp.float32)],
        compiler_params=_PARAMS,
    )(sprev, wlt, u, wgt, bg)


def _tail_call(u, sprev, wlt):
    n, k = u.shape
    pk = sprev.shape[1]
    return pl.pallas_call(
        functools.partial(_tail_body, inv_n=1.0 / n),
        grid=(n // _TM,),
        in_specs=[_full((1, pk)), _full((pk, k)),
                  pl.BlockSpec((_TM, k), lambda i: (i, 0))],
        out_specs=pl.BlockSpec((_TM, k), lambda i: (i, 0)),
        out_shape=jax.ShapeDtypeStruct((n, k), jnp.float32),
        compiler_params=_PARAMS,
    )(sprev, wlt, u)


def kernel(x, Wg1, bg1, Wl1, Wg2, bg2, Wl2, Wg3, bg3, Wl3, Wg4, bg4, Wl4):
    bf = jnp.bfloat16
    wg = [w.T.astype(bf) for w in (Wg1, Wg2, Wg3, Wg4)]
    wl = [w.T.astype(bf) for w in (Wl1, Wl2, Wl3, Wl4)]
    bg = [b.reshape(1, -1).astype(jnp.float32) for b in (bg1, bg2, bg3, bg4)]

    u1, s0 = _head_call(x, wg[0], bg[0])
    u2, s1 = _mid_call(u1, s0, wl[0], wg[1], bg[1], bf)
    u3, s2 = _mid_call(u2, s1, wl[1], wg[2], bg[2], bf)
    u4, s3 = _mid_call(u3, s2, wl[2], wg[3], bg[3], jnp.float32)
    return _tail_call(u4, s3, wl[3])
